# baseline (device time: 55220 ns/iter reference)
import jax
import jax.numpy as jnp
from jax import lax
from jax.experimental import pallas as pl
from jax.experimental.pallas import tpu as pltpu

N_DEV = 4
QUARTERS = 4


def kernel(x, w_mat):
    m_per, k = x.shape
    _, n = w_mat.shape
    n_per = n // N_DEV
    m_q = m_per // QUARTERS

    def body(x_hbm, w_hbm, out_ref,
             x_stage, x_bf, w_stage, w_bf, y_send, recv_buf,
             x_sems, w_sems, send_sems, recv_sems):
        my = lax.axis_index("i")

        barrier_sem = pltpu.get_barrier_semaphore()
        for off in range(1, N_DEV):
            peer = lax.rem(my + off, N_DEV)
            pl.semaphore_signal(
                barrier_sem, inc=1,
                device_id=(peer,), device_id_type=pl.DeviceIdType.MESH,
            )
        pl.semaphore_wait(barrier_sem, N_DEV - 1)

        def block_of(t):
            return lax.rem(my + 1 + t, N_DEV)

        def x_copy(c):
            return pltpu.make_async_copy(
                x_hbm.at[pl.ds(c * m_q, m_q), :],
                x_stage.at[c % 2],
                x_sems.at[c % 2],
            )

        def w_copy(t):
            return pltpu.make_async_copy(
                w_hbm.at[:, pl.ds(block_of(t) * n_per, n_per)],
                w_stage.at[t % 2],
                w_sems.at[t % 2],
            )

        def x_land(c):
            x_copy(c).wait()
            x_bf[pl.ds(c * m_q, m_q), :] = x_stage[c % 2].astype(jnp.bfloat16)
            if c + 2 < QUARTERS:
                x_copy(c + 2).start()

        def q_rdma(t, q, dev):
            return pltpu.make_async_remote_copy(
                src_ref=y_send.at[t, pl.ds(q * m_q, m_q), :],
                dst_ref=recv_buf.at[t, pl.ds(q * m_q, m_q), :],
                send_sem=send_sems.at[t, q],
                recv_sem=recv_sems.at[t, q],
                device_id=(dev,),
                device_id_type=pl.DeviceIdType.MESH,
            )

        x_copy(0).start()
        x_copy(1).start()
        w_copy(0).start()
        w_copy(1).start()

        x_land(0)

        for t in range(N_DEV):
            w_copy(t).wait()
            w_bf[t % 2] = w_stage[t % 2].astype(jnp.bfloat16)
            if t + 2 < N_DEV:
                w_copy(t + 2).start()

            for q in range(QUARTERS):
                yq = jnp.dot(
                    x_bf[pl.ds(q * m_q, m_q), :], w_bf[t % 2],
                    preferred_element_type=jnp.float32)
                if t < N_DEV - 1:
                    y_send[t, pl.ds(q * m_q, m_q), :] = (
                        yq.astype(jnp.bfloat16))
                    q_rdma(t, q, block_of(t)).start()
                else:
                    out_ref[pl.ds(my * m_per + q * m_q, m_q), :] = yq
                if t == 0 and q + 1 < QUARTERS:
                    x_land(q + 1)

        for t in range(N_DEV - 1):
            src_dev = lax.rem(my + N_DEV - 1 - t, N_DEV)
            for q in range(QUARTERS):
                q_rdma(t, q, src_dev).wait()
            out_ref[pl.ds(src_dev * m_per, m_per), :] = (
                recv_buf[t, :, :].astype(jnp.float32))

    return pl.pallas_call(
        body,
        out_shape=jax.ShapeDtypeStruct((N_DEV * m_per, n_per), jnp.float32),
        in_specs=[
            pl.BlockSpec(memory_space=pltpu.MemorySpace.HBM),
            pl.BlockSpec(memory_space=pltpu.MemorySpace.HBM),
        ],
        out_specs=pl.BlockSpec(memory_space=pltpu.VMEM),
        scratch_shapes=[
            pltpu.VMEM((2, m_q, k), jnp.float32),
            pltpu.VMEM((m_per, k), jnp.bfloat16),
            pltpu.VMEM((2, k, n_per), jnp.float32),
            pltpu.VMEM((2, k, n_per), jnp.bfloat16),
            pltpu.VMEM((N_DEV - 1, m_per, n_per), jnp.bfloat16),
            pltpu.VMEM((N_DEV - 1, m_per, n_per), jnp.bfloat16),
            pltpu.SemaphoreType.DMA((2,)),
            pltpu.SemaphoreType.DMA((2,)),
            pltpu.SemaphoreType.DMA((N_DEV - 1, QUARTERS)),
            pltpu.SemaphoreType.DMA((N_DEV - 1, QUARTERS)),
        ],
        compiler_params=pltpu.CompilerParams(
            collective_id=0,
            vmem_limit_bytes=60 * 1024 * 1024,
        ),
    )(x, w_mat)
